# baseline (speedup 1.0000x reference)
import functools

import jax
import jax.numpy as jnp
from jax import lax
from jax.experimental import pallas as pl
from jax.experimental.pallas import tpu as pltpu
from jax.experimental.pallas import tpu_sc as plsc

ROWS = 16384
COLS = 26
D = 32
B = ROWS * COLS
NW = 32
LANES = 128
EPR = LANES // D

TBL_COLS = 1000001
NBLK = 7812
TAIL_COL = NBLK * LANES
SROWS = (NBLK + 1) * D

CHUNK = 128
NCHUNK_W = B // (NW * CHUNK)

_mesh = plsc.VectorSubcoreMesh(core_axis_name="c", subcore_axis_name="s")
_params = pltpu.CompilerParams(
    use_tc_tiling_on_sc=True, needs_layout_passes=False
)


def _wid():
    return lax.axis_index("s") * 2 + lax.axis_index("c")


def _iota16():
    return lax.iota(jnp.int32, 16)


@functools.partial(
    pl.kernel,
    out_type=jax.ShapeDtypeStruct((SROWS, LANES), jnp.float32),
    mesh=_mesh,
    compiler_params=_params,
    scratch_types=[
        pltpu.VMEM((2, D, LANES), jnp.float32),
        pltpu.VMEM((2, D, LANES), jnp.float32),
        pltpu.SemaphoreType.DMA((2,)),
        pltpu.SemaphoreType.DMA((2,)),
    ],
)
def _transpose_kernel(table_hbm, tail_hbm, scratch_hbm, tin, tout, isem, osem):
    w = _wid()
    lo = w * NBLK // NW
    hi = (w + 1) * NBLK // NW

    def start_in(c, p):
        pltpu.async_copy(
            table_hbm.at[:, pl.ds(c * LANES, LANES)], tin.at[p], isem.at[p]
        )

    def wait_in(p):
        pltpu.make_async_copy(
            table_hbm.at[:, pl.ds(0, LANES)], tin.at[p], isem.at[p]
        ).wait()

    def start_out(c, p):
        pltpu.async_copy(
            tout.at[p], scratch_hbm.at[pl.ds(c * D, D)], osem.at[p]
        )

    def wait_out(p):
        pltpu.make_async_copy(
            tout.at[p], scratch_hbm.at[pl.ds(0, D)], osem.at[p]
        ).wait()

    def shuffle(p):
        it = _iota16()
        for g in range(8):
            rows = it + 16 * (g % 2)
            for j in range(D):
                cols = jnp.full((16,), EPR * j + g // 2, jnp.int32)
                tout[p, j, pl.ds(16 * g, 16)] = plsc.load_gather(
                    tin.at[p], [rows, cols]
                )

    start_in(lo, 0)

    def body(k, _):
        c = lo + k
        p = lax.rem(k, 2)
        wait_in(p)

        @pl.when(c + 1 < hi)
        def _():
            start_in(c + 1, 1 - p)

        @pl.when(k >= 2)
        def _():
            wait_out(p)

        shuffle(p)
        start_out(c, p)
        return 0

    lax.fori_loop(0, hi - lo, body, 0)
    wait_out(lax.rem(hi - lo - 2, 2))
    wait_out(lax.rem(hi - lo - 1, 2))

    @pl.when(w == NW - 1)
    def _():
        pltpu.async_copy(
            tail_hbm, tin.at[0, pl.ds(0, 16)], isem.at[0]
        ).wait()
        pltpu.async_copy(
            tin.at[0, pl.ds(0, 16)],
            scratch_hbm.at[pl.ds(NBLK * D, 16)],
            osem.at[0],
        ).wait()


@functools.partial(
    pl.kernel,
    out_type=jax.ShapeDtypeStruct((COLS, D, ROWS), jnp.float32),
    mesh=_mesh,
    compiler_params=_params,
    scratch_types=[
        pltpu.VMEM((4, CHUNK), jnp.int32),
        pltpu.VMEM((2, CHUNK), jnp.int32),
        pltpu.VMEM((2, CHUNK), jnp.int32),
        pltpu.VMEM((2, CHUNK, LANES), jnp.float32),
        pltpu.VMEM((2, D, CHUNK), jnp.float32),
        pltpu.SemaphoreType.DMA((4,)),
        pltpu.SemaphoreType.DMA((2,)),
        pltpu.SemaphoreType.DMA((2,)),
    ],
)
def _gather_kernel(idx_hbm, scratch_hbm, out_hbm, idxv, jl, rb, gbuf, vout,
                   qsem, gsem, osem):
    w = _wid()
    u0 = w * NCHUNK_W

    def unit(k):
        u = u0 + k
        return lax.div(u, ROWS // CHUNK), lax.rem(u, ROWS // CHUNK)

    def start_idx(k, q):
        b, ablk = unit(k)
        pltpu.async_copy(
            idx_hbm.at[b, pl.ds(ablk * CHUNK, CHUNK)], idxv.at[q], qsem.at[q]
        )

    def wait_idx(q):
        pltpu.make_async_copy(
            idx_hbm.at[0, pl.ds(0, CHUNK)], idxv.at[q], qsem.at[q]
        ).wait()

    def compute(q, p):
        for t in range(CHUNK // 16):
            v = idxv[q, pl.ds(16 * t, 16)]
            jl[p, pl.ds(16 * t, 16)] = v >> 2
            rb[p, pl.ds(16 * t, 16)] = (v & 3) * D

    def start_gather(p):
        pltpu.async_copy(scratch_hbm.at[jl.at[p]], gbuf.at[p], gsem.at[p])

    def wait_gather(p):
        pltpu.make_async_copy(
            scratch_hbm.at[jl.at[p]], gbuf.at[p], gsem.at[p]
        ).wait()

    def start_out(k, p):
        b, ablk = unit(k)
        pltpu.async_copy(
            vout.at[p], out_hbm.at[b, :, pl.ds(ablk * CHUNK, CHUNK)], osem.at[p]
        )

    def wait_out(p):
        pltpu.make_async_copy(
            vout.at[p], out_hbm.at[0, :, pl.ds(0, CHUNK)], osem.at[p]
        ).wait()

    def shuffle(p):
        it = _iota16()
        for g in range(8):
            rows = it + 16 * g
            rbase = rb[p, pl.ds(16 * g, 16)]
            for d in range(D):
                vout[p, d, pl.ds(16 * g, 16)] = plsc.load_gather(
                    gbuf.at[p], [rows, rbase + d]
                )

    start_idx(0, 0)
    start_idx(1, 1)
    wait_idx(0)
    compute(0, 0)
    start_gather(0)

    def body(k, _):
        p = lax.rem(k, 2)

        @pl.when(k + 2 < NCHUNK_W)
        def _():
            start_idx(k + 2, lax.rem(k + 2, 4))

        @pl.when(k + 1 < NCHUNK_W)
        def _():
            wait_idx(lax.rem(k + 1, 4))
            compute(lax.rem(k + 1, 4), 1 - p)
            start_gather(1 - p)

        wait_gather(p)

        @pl.when(k >= 2)
        def _():
            wait_out(p)

        shuffle(p)
        start_out(k, p)
        return 0

    lax.fori_loop(0, NCHUNK_W, body, 0)
    wait_out(0)
    wait_out(1)


def kernel(inputs, embeddings):
    idx_t = inputs.T
    table_t = embeddings.T
    tail16 = embeddings[TAIL_COL:TAIL_COL + 64].reshape(16, LANES)
    scratch = _transpose_kernel(table_t, tail16)
    out_t = _gather_kernel(idx_t, scratch)
    return out_t.transpose(2, 0, 1)


# speedup vs baseline: 1.8625x; 1.8625x over previous
import functools

import jax
import jax.numpy as jnp
from jax import lax
from jax.experimental import pallas as pl
from jax.experimental.pallas import tpu as pltpu
from jax.experimental.pallas import tpu_sc as plsc

ROWS = 16384
COLS = 26
D = 32
B = ROWS * COLS
NW = 32
LANES = 128
EPR = LANES // D

TBL_COLS = 1000001
NBLK = 7812
TAIL_COL = NBLK * LANES
SROWS = (NBLK + 1) * D

CHUNK = 128
NCHUNK_W = B // (NW * CHUNK)
NB = 4

_mesh = plsc.VectorSubcoreMesh(core_axis_name="c", subcore_axis_name="s")
_params = pltpu.CompilerParams(
    use_tc_tiling_on_sc=True, needs_layout_passes=False
)


def _wid():
    return lax.axis_index("s") * 2 + lax.axis_index("c")


def _iota16():
    return lax.iota(jnp.int32, 16)


@functools.partial(
    pl.kernel,
    out_type=jax.ShapeDtypeStruct((SROWS, LANES), jnp.float32),
    mesh=_mesh,
    compiler_params=_params,
    scratch_types=[
        pltpu.VMEM((NB, D, LANES), jnp.float32),
        pltpu.VMEM((NB, D, LANES), jnp.float32),
        pltpu.SemaphoreType.DMA((NB,)),
        pltpu.SemaphoreType.DMA((NB,)),
    ],
)
def _transpose_kernel(table_hbm, tail_hbm, scratch_hbm, tin, tout, isem, osem):
    w = _wid()
    lo = w * NBLK // NW
    hi = (w + 1) * NBLK // NW

    def start_in(c, p):
        pltpu.async_copy(
            table_hbm.at[:, pl.ds(c * LANES, LANES)], tin.at[p], isem.at[p]
        )

    def wait_in(p):
        pltpu.make_async_copy(
            table_hbm.at[:, pl.ds(0, LANES)], tin.at[p], isem.at[p]
        ).wait()

    def start_out(c, p):
        pltpu.async_copy(
            tout.at[p], scratch_hbm.at[pl.ds(c * D, D)], osem.at[p]
        )

    def wait_out(p):
        pltpu.make_async_copy(
            tout.at[p], scratch_hbm.at[pl.ds(0, D)], osem.at[p]
        ).wait()

    def shuffle(p):
        it = _iota16()
        rows = [it + 16 * (g % 2) for g in range(8)]

        @plsc.parallel_loop(0, D, 1, unroll=4)
        def _(j):
            for g in range(8):
                cols = jnp.full((16,), EPR * j + g // 2, jnp.int32)
                tout[p, j, pl.ds(16 * g, 16)] = plsc.load_gather(
                    tin.at[p], [rows[g], cols]
                )

    for r in range(NB - 1):
        start_in(lo + r, r)

    def body(k, _):
        c = lo + k
        p = lax.rem(k, NB)
        wait_in(p)

        @pl.when(c + NB - 1 < hi)
        def _():
            start_in(c + NB - 1, lax.rem(k + NB - 1, NB))

        @pl.when(k >= NB)
        def _():
            wait_out(p)

        shuffle(p)
        start_out(c, p)
        return 0

    lax.fori_loop(0, hi - lo, body, 0)
    for r in range(NB):
        wait_out(lax.rem(hi - lo - NB + r, NB))

    @pl.when(w == NW - 1)
    def _():
        pltpu.async_copy(
            tail_hbm, tin.at[0, pl.ds(0, 16)], isem.at[0]
        ).wait()
        pltpu.async_copy(
            tin.at[0, pl.ds(0, 16)],
            scratch_hbm.at[pl.ds(NBLK * D, 16)],
            osem.at[0],
        ).wait()


@functools.partial(
    pl.kernel,
    out_type=jax.ShapeDtypeStruct((COLS, D, ROWS), jnp.float32),
    mesh=_mesh,
    compiler_params=_params,
    scratch_types=[
        pltpu.VMEM((NB, CHUNK), jnp.int32),
        pltpu.VMEM((NB, CHUNK), jnp.int32),
        pltpu.VMEM((NB, CHUNK), jnp.int32),
        pltpu.VMEM((NB, CHUNK, LANES), jnp.float32),
        pltpu.VMEM((NB, D, CHUNK), jnp.float32),
        pltpu.SemaphoreType.DMA((NB,)),
        pltpu.SemaphoreType.DMA((NB,)),
        pltpu.SemaphoreType.DMA((NB,)),
    ],
)
def _gather_kernel(idx_hbm, scratch_hbm, out_hbm, idxv, jl, rb, gbuf, vout,
                   qsem, gsem, osem):
    w = _wid()
    u0 = w * NCHUNK_W

    def unit(k):
        u = u0 + k
        return lax.div(u, ROWS // CHUNK), lax.rem(u, ROWS // CHUNK)

    def start_idx(k, q):
        b, ablk = unit(k)
        pltpu.async_copy(
            idx_hbm.at[b, pl.ds(ablk * CHUNK, CHUNK)], idxv.at[q], qsem.at[q]
        )

    def wait_idx(q):
        pltpu.make_async_copy(
            idx_hbm.at[0, pl.ds(0, CHUNK)], idxv.at[q], qsem.at[q]
        ).wait()

    def compute(q):
        for t in range(CHUNK // 16):
            v = idxv[q, pl.ds(16 * t, 16)]
            jl[q, pl.ds(16 * t, 16)] = v >> 2
            rb[q, pl.ds(16 * t, 16)] = (v & 3) * D

    def start_gather(p):
        pltpu.async_copy(scratch_hbm.at[jl.at[p]], gbuf.at[p], gsem.at[p])

    def wait_gather(p):
        pltpu.make_async_copy(
            scratch_hbm.at[jl.at[p]], gbuf.at[p], gsem.at[p]
        ).wait()

    def start_out(k, p):
        b, ablk = unit(k)
        pltpu.async_copy(
            vout.at[p], out_hbm.at[b, :, pl.ds(ablk * CHUNK, CHUNK)], osem.at[p]
        )

    def wait_out(p):
        pltpu.make_async_copy(
            vout.at[p], out_hbm.at[0, :, pl.ds(0, CHUNK)], osem.at[p]
        ).wait()

    def shuffle(p):
        it = _iota16()
        rows = [it + 16 * g for g in range(8)]
        rbase = [rb[p, pl.ds(16 * g, 16)] for g in range(8)]

        @plsc.parallel_loop(0, D, 1, unroll=4)
        def _(d):
            for g in range(8):
                vout[p, d, pl.ds(16 * g, 16)] = plsc.load_gather(
                    gbuf.at[p], [rows[g], rbase[g] + d]
                )

    for r in range(NB):
        start_idx(r, r)
    for r in range(2):
        wait_idx(r)
        compute(r)
        start_gather(r)

    def body(k, _):
        p = lax.rem(k, NB)

        @pl.when(k + NB < NCHUNK_W)
        def _():
            start_idx(k + NB, p)

        @pl.when(k + 2 < NCHUNK_W)
        def _():
            q = lax.rem(k + 2, NB)
            wait_idx(q)
            compute(q)
            start_gather(q)

        wait_gather(p)

        @pl.when(k >= NB)
        def _():
            wait_out(p)

        shuffle(p)
        start_out(k, p)
        return 0

    lax.fori_loop(0, NCHUNK_W, body, 0)
    for r in range(NB):
        wait_out(r)


def kernel(inputs, embeddings):
    idx_t = inputs.T
    table_t = embeddings.T
    tail16 = embeddings[TAIL_COL:TAIL_COL + 64].reshape(16, LANES)
    scratch = _transpose_kernel(table_t, tail16)
    out_t = _gather_kernel(idx_t, scratch)
    return out_t.transpose(2, 0, 1)
